# Initial kernel scaffold; baseline (speedup 1.0000x reference)
import jax
import jax.numpy as jnp
from jax.experimental import pallas as pl


def kernel(edu_reps, speaker_ids, batch_speaker_maps, W_sq, b_sq, W_cq, b_cq, W_sp, b_sp, W_cp, b_cp, W_ih, W_hh, b_ih, b_hh):
    raise NotImplementedError("write your pallas kernel here")


# speedup vs baseline: 51.4630x; 51.4630x over previous
import functools

import jax
import jax.numpy as jnp
from jax import lax
from jax.experimental import pallas as pl
from jax.experimental.pallas import tpu as pltpu

_B, _T, _D, _S, _K = 4, 256, 768, 8, 5
_NEG = -3e38


def _dot_t(x, w, precision=None):
    return lax.dot_general(x, w, (((1,), (1,)), ((), ())),
                           precision=precision,
                           preferred_element_type=jnp.float32)


def _stage1_body(x_ref, wsq_ref, bsq_ref, wcq_ref, bcq_ref, wsp_ref, bsp_ref,
                 wcp_ref, bcp_ref, wih_ref, bih_ref, gi_ref):
    x = x_ref[0]
    qs = _dot_t(x, wsq_ref[...]) + bsq_ref[...]
    qc = _dot_t(x, wcq_ref[...]) + bcq_ref[...]
    sp = _dot_t(x, wsp_ref[...]) + bsp_ref[...]
    cp = _dot_t(x, wcp_ref[...]) + bcp_ref[...]

    scores = _dot_t(qs, sp) + _dot_t(qc, cp)

    row = lax.broadcasted_iota(jnp.int32, (_T, _T), 0)
    col = lax.broadcasted_iota(jnp.int32, (_T, _T), 1)
    masked = jnp.where(col < row, scores, _NEG)

    wsel = jnp.zeros((_T, _T), jnp.float32)
    for _ in range(_K):
        rowmax = jnp.max(masked, axis=1, keepdims=True)
        cand = jnp.where(masked == rowmax, col, _T)
        selj = jnp.min(cand, axis=1, keepdims=True)
        pick = col == selj
        valid = rowmax > jnp.float32(-1e38)
        wsel = wsel + jnp.where(jnp.logical_and(pick, valid), 1.0, 0.0)
        masked = jnp.where(pick, _NEG, masked)

    t_idx = row[:, :1].astype(jnp.float32)
    cnt = jnp.clip(t_idx, 1.0, float(_K))
    summary = jax.lax.dot(wsel, x, precision=lax.Precision.HIGHEST,
                          preferred_element_type=jnp.float32) / cnt

    gi_ref[0] = _dot_t(summary, wih_ref[...]) + bih_ref[...]


def _stage2_body(gi_ref, spk_ref, whh_ref, bhh_ref, out_ref, mem_ref):
    mem_ref[...] = jnp.zeros((_B * _S, _D), jnp.float32)

    def step(t, carry):
        rows = [mem_ref[pl.ds(_S * b + spk_ref[b, t], 1), :]
                for b in range(_B)]
        h = jnp.concatenate(rows, axis=0)
        gh = _dot_t(h, whh_ref[...]) + bhh_ref[...]
        git = gi_ref[:, pl.ds(t, 1), :].reshape(_B, 3 * _D)
        r = jax.nn.sigmoid(git[:, :_D] + gh[:, :_D])
        z = jax.nn.sigmoid(git[:, _D:2 * _D] + gh[:, _D:2 * _D])
        n = jnp.tanh(git[:, 2 * _D:] + r * gh[:, 2 * _D:])
        hn = (1.0 - z) * n + z * h
        for b in range(_B):
            mem_ref[pl.ds(_S * b + spk_ref[b, t], 1), :] = hn[b:b + 1, :]
        return carry

    lax.fori_loop(1, _T, step, 0)
    for b in range(_B):
        out_ref[b] = mem_ref[_S * b:_S * (b + 1), :]


def kernel(edu_reps, speaker_ids, batch_speaker_maps, W_sq, b_sq, W_cq, b_cq,
           W_sp, b_sp, W_cp, b_cp, W_ih, W_hh, b_ih, b_hh):
    del batch_speaker_maps
    x = edu_reps.astype(jnp.float32)
    spk = speaker_ids.astype(jnp.int32)
    row2 = lambda v: v.reshape(1, -1).astype(jnp.float32)

    full = lambda shape: pl.BlockSpec(shape, lambda b: (0,) * len(shape))
    gi = pl.pallas_call(
        _stage1_body,
        grid=(_B,),
        in_specs=[
            pl.BlockSpec((1, _T, _D), lambda b: (b, 0, 0)),
            full((_D, _D)), full((1, _D)),
            full((_D, _D)), full((1, _D)),
            full((_D, _D)), full((1, _D)),
            full((_D, _D)), full((1, _D)),
            full((3 * _D, _D)), full((1, 3 * _D)),
        ],
        out_specs=pl.BlockSpec((1, _T, 3 * _D), lambda b: (b, 0, 0)),
        out_shape=jax.ShapeDtypeStruct((_B, _T, 3 * _D), jnp.float32),
    )(x, W_sq, row2(b_sq), W_cq, row2(b_cq), W_sp, row2(b_sp),
      W_cp, row2(b_cp), W_ih, row2(b_ih))

    mem = pl.pallas_call(
        _stage2_body,
        in_specs=[
            pl.BlockSpec(memory_space=pltpu.VMEM),
            pl.BlockSpec(memory_space=pltpu.SMEM),
            pl.BlockSpec(memory_space=pltpu.VMEM),
            pl.BlockSpec(memory_space=pltpu.VMEM),
        ],
        out_specs=pl.BlockSpec(memory_space=pltpu.VMEM),
        out_shape=jax.ShapeDtypeStruct((_B, _S, _D), jnp.float32),
        scratch_shapes=[pltpu.VMEM((_B * _S, _D), jnp.float32)],
    )(gi, spk, W_hh, row2(b_hh))

    return mem


# speedup vs baseline: 124.8708x; 2.4264x over previous
import functools

import jax
import jax.numpy as jnp
from jax import lax
from jax.experimental import pallas as pl
from jax.experimental.pallas import tpu as pltpu

_B, _T, _D, _S, _K = 4, 256, 768, 8, 5
_NEG = -3e38


def _dot_t(x, w, precision=None):
    return lax.dot_general(x, w, (((1,), (1,)), ((), ())),
                           precision=precision,
                           preferred_element_type=jnp.float32)


def _stage1_body(x_ref, wsq_ref, bsq_ref, wcq_ref, bcq_ref, wsp_ref, bsp_ref,
                 wcp_ref, bcp_ref, wih_ref, bih_ref, gi_ref):
    x = x_ref[0]
    qs = _dot_t(x, wsq_ref[...]) + bsq_ref[...]
    qc = _dot_t(x, wcq_ref[...]) + bcq_ref[...]
    sp = _dot_t(x, wsp_ref[...]) + bsp_ref[...]
    cp = _dot_t(x, wcp_ref[...]) + bcp_ref[...]

    scores = _dot_t(qs, sp) + _dot_t(qc, cp)

    row = lax.broadcasted_iota(jnp.int32, (_T, _T), 0)
    col = lax.broadcasted_iota(jnp.int32, (_T, _T), 1)
    masked = jnp.where(col < row, scores, _NEG)

    wsel = jnp.zeros((_T, _T), jnp.float32)
    for _ in range(_K):
        rowmax = jnp.max(masked, axis=1, keepdims=True)
        cand = jnp.where(masked == rowmax, col, _T)
        selj = jnp.min(cand, axis=1, keepdims=True)
        pick = col == selj
        valid = rowmax > jnp.float32(-1e38)
        wsel = wsel + jnp.where(jnp.logical_and(pick, valid), 1.0, 0.0)
        masked = jnp.where(pick, _NEG, masked)

    t_idx = row[:, :1].astype(jnp.float32)
    cnt = jnp.clip(t_idx, 1.0, float(_K))
    summary = jax.lax.dot(wsel, x, precision=lax.Precision.HIGHEST,
                          preferred_element_type=jnp.float32) / cnt

    gi_ref[0] = _dot_t(summary, wih_ref[...]) + bih_ref[...]


_C = _B * _S


def _stage2_body(gi_ref, spk_ref, whh_ref, bhh_ref, out_ref, pt_ref):
    t_row = lax.broadcasted_iota(jnp.int32, (1, _T), 1)
    tril = (lax.broadcasted_iota(jnp.int32, (_T, _T), 0) <
            lax.broadcasted_iota(jnp.int32, (_T, _T), 1))
    u_mat = tril.astype(jnp.float32)
    r_iota = lax.broadcasted_iota(jnp.int32, (_T, 1), 0)
    s_iota = lax.broadcasted_iota(jnp.int32, (_S, 1), 0)

    pts, lts = [], []
    for b in range(_B):
        spk_row = spk_ref[b:b + 1, :]
        o_b = jnp.where((s_iota == spk_row) & (t_row > 0), 1.0, 0.0)
        cnt = jax.lax.dot(o_b, u_mat,
                          preferred_element_type=jnp.float32)
        rank = jnp.sum(o_b * cnt, axis=0, keepdims=True)
        r_oh = jnp.where(r_iota.astype(jnp.float32) == rank, 1.0, 0.0)
        a_b = o_b * t_row.astype(jnp.float32)
        pts.append(_dot_t(r_oh, a_b))
        lts.append(_dot_t(jnp.ones((1, _T), jnp.float32), o_b))
    pt_ref[...] = jnp.concatenate(pts, axis=1)
    lt = jnp.concatenate(lts, axis=1)

    c_iota = lax.broadcasted_iota(jnp.int32, (1, _C), 1)
    boff = ((c_iota // _S) * _T).astype(jnp.float32)
    eye_c = jnp.where(
        lax.broadcasted_iota(jnp.int32, (_C, _C), 0) ==
        lax.broadcasted_iota(jnp.int32, (_C, _C), 1), 1.0, 0.0)
    j_row = lax.broadcasted_iota(jnp.int32, (1, _B * _T), 1).astype(
        jnp.float32)
    gi2 = gi_ref[...]
    whh = whh_ref[...]
    bhh = bhh_ref[...]
    lmax = jnp.max(lt).astype(jnp.int32)

    def step(i, h):
        fi_row = pt_ref[pl.ds(i, 1), :] + boff
        fi_col = _dot_t(eye_c, fi_row)
        g = jnp.where(j_row == fi_col, 1.0, 0.0)
        git = jax.lax.dot(g, gi2,
                          preferred_element_type=jnp.float32)
        gh = _dot_t(h, whh) + bhh
        r = jax.nn.sigmoid(git[:, :_D] + gh[:, :_D])
        z = jax.nn.sigmoid(git[:, _D:2 * _D] + gh[:, _D:2 * _D])
        n = jnp.tanh(git[:, 2 * _D:] + r * gh[:, 2 * _D:])
        hn = (1.0 - z) * n + z * h
        m_col = _dot_t(eye_c, jnp.where(
            i.astype(jnp.float32) < lt, 1.0, 0.0))
        return m_col * hn + (1.0 - m_col) * h

    h = lax.fori_loop(0, lmax, step,
                      jnp.zeros((_C, _D), jnp.float32))
    out_ref[...] = h


def kernel(edu_reps, speaker_ids, batch_speaker_maps, W_sq, b_sq, W_cq, b_cq,
           W_sp, b_sp, W_cp, b_cp, W_ih, W_hh, b_ih, b_hh):
    del batch_speaker_maps
    x = edu_reps.astype(jnp.float32)
    spk = speaker_ids.astype(jnp.int32)
    row2 = lambda v: v.reshape(1, -1).astype(jnp.float32)

    full = lambda shape: pl.BlockSpec(shape, lambda b: (0,) * len(shape))
    gi = pl.pallas_call(
        _stage1_body,
        grid=(_B,),
        in_specs=[
            pl.BlockSpec((1, _T, _D), lambda b: (b, 0, 0)),
            full((_D, _D)), full((1, _D)),
            full((_D, _D)), full((1, _D)),
            full((_D, _D)), full((1, _D)),
            full((_D, _D)), full((1, _D)),
            full((3 * _D, _D)), full((1, 3 * _D)),
        ],
        out_specs=pl.BlockSpec((1, _T, 3 * _D), lambda b: (b, 0, 0)),
        out_shape=jax.ShapeDtypeStruct((_B, _T, 3 * _D), jnp.float32),
    )(x, W_sq, row2(b_sq), W_cq, row2(b_cq), W_sp, row2(b_sp),
      W_cp, row2(b_cp), W_ih, row2(b_ih))

    mem = pl.pallas_call(
        _stage2_body,
        in_specs=[
            pl.BlockSpec(memory_space=pltpu.VMEM),
            pl.BlockSpec(memory_space=pltpu.VMEM),
            pl.BlockSpec(memory_space=pltpu.VMEM),
            pl.BlockSpec(memory_space=pltpu.VMEM),
        ],
        out_specs=pl.BlockSpec(memory_space=pltpu.VMEM),
        out_shape=jax.ShapeDtypeStruct((_C, _D), jnp.float32),
        scratch_shapes=[pltpu.VMEM((_T, _C), jnp.float32)],
    )(gi.reshape(_B * _T, 3 * _D), spk, W_hh, row2(b_hh))

    return mem.reshape(_B, _S, _D)


# speedup vs baseline: 167.0028x; 1.3374x over previous
import jax
import jax.numpy as jnp
from jax import lax
from jax.experimental import pallas as pl
from jax.experimental.pallas import tpu as pltpu

_B, _T, _D, _S, _K = 4, 256, 768, 8, 5
_C = _B * _S
_NEG = -3e38


def _dot_t(x, w):
    return lax.dot_general(x, w, (((1,), (1,)), ((), ())),
                           preferred_element_type=jnp.float32)


def _ptable_body(spk_ref, pt_ref, l_ref, lmax_ref):
    t_row = lax.broadcasted_iota(jnp.int32, (1, _T), 1)
    upper = jnp.where(
        lax.broadcasted_iota(jnp.int32, (_T, _T), 0) <
        lax.broadcasted_iota(jnp.int32, (_T, _T), 1), 1.0, 0.0)
    r_iota = lax.broadcasted_iota(jnp.int32, (_T, 1), 0)
    s_iota = lax.broadcasted_iota(jnp.int32, (_S, 1), 0)

    pts, lts = [], []
    for b in range(_B):
        spk_row = spk_ref[b:b + 1, :]
        o_b = jnp.where((s_iota == spk_row) & (t_row > 0), 1.0, 0.0)
        cnt = jax.lax.dot(o_b, upper,
                          preferred_element_type=jnp.float32)
        rank = jnp.sum(o_b * cnt, axis=0, keepdims=True)
        r_oh = jnp.where(r_iota.astype(jnp.float32) == rank, 1.0, 0.0)
        a_b = o_b * t_row.astype(jnp.float32)
        pts.append(_dot_t(r_oh, a_b))
        lts.append(_dot_t(jnp.ones((1, _T), jnp.float32), o_b))
    pt_ref[...] = jnp.concatenate(pts, axis=1).astype(jnp.int32)
    lt = jnp.concatenate(lts, axis=1)
    l_ref[...] = lt
    lmax_ref[...] = jnp.max(lt, axis=1, keepdims=True).astype(jnp.int32)


def _fused_body(x_ref, wqp_ref, bqp_ref, wih_ref, bih_ref, whh_ref, bhh_ref,
                l_ref, pt_ref, lmax_ref, out_ref, gi_ref, xg_ref):
    row = lax.broadcasted_iota(jnp.int32, (_T, _T), 0)
    col = lax.broadcasted_iota(jnp.int32, (_T, _T), 1)
    causal = col < row
    t_col = row[:, :1].astype(jnp.float32)
    cnt = jnp.clip(t_col, 1.0, float(_K))

    for b in range(_B):
        x = x_ref[b]
        qp = _dot_t(x, wqp_ref[...]) + bqp_ref[...]
        scores = _dot_t(qp[:, :2 * _D], qp[:, 2 * _D:])
        masked = jnp.where(causal, scores, _NEG)

        wsel = jnp.zeros((_T, _T), jnp.float32)
        for _ in range(_K):
            rowmax = jnp.max(masked, axis=1, keepdims=True)
            cand = jnp.where(masked == rowmax, col, _T)
            selj = jnp.min(cand, axis=1, keepdims=True)
            pick = col == selj
            valid = rowmax > jnp.float32(-1e38)
            wsel = wsel + jnp.where(jnp.logical_and(pick, valid), 1.0, 0.0)
            masked = jnp.where(pick, _NEG, masked)

        summary = jax.lax.dot(wsel, x, precision=lax.Precision.HIGHEST,
                              preferred_element_type=jnp.float32) / cnt
        gi_ref[_T * b:_T * (b + 1), :] = (
            _dot_t(summary, wih_ref[...]) + bih_ref[...])

    whh = whh_ref[...]
    bhh = bhh_ref[...]
    lcol = l_ref[...]
    lmax = lmax_ref[0, 0]

    def step(i, h):
        for c in range(_C):
            xg_ref[c:c + 1, :] = gi_ref[
                pl.ds((c // _S) * _T + pt_ref[i, c], 1), :]
        git = xg_ref[...]
        gh = _dot_t(h, whh) + bhh
        r = jax.nn.sigmoid(git[:, :_D] + gh[:, :_D])
        z = jax.nn.sigmoid(git[:, _D:2 * _D] + gh[:, _D:2 * _D])
        n = jnp.tanh(git[:, 2 * _D:] + r * gh[:, 2 * _D:])
        hn = (1.0 - z) * n + z * h
        m = jnp.where(i.astype(jnp.float32) < lcol, 1.0, 0.0)
        return m * hn + (1.0 - m) * h

    h = lax.fori_loop(0, lmax, step, jnp.zeros((_C, _D), jnp.float32))
    out_ref[...] = h


def kernel(edu_reps, speaker_ids, batch_speaker_maps, W_sq, b_sq, W_cq, b_cq,
           W_sp, b_sp, W_cp, b_cp, W_ih, W_hh, b_ih, b_hh):
    del batch_speaker_maps
    x = edu_reps.astype(jnp.float32)
    spk = speaker_ids.astype(jnp.int32)
    row2 = lambda v: v.reshape(1, -1).astype(jnp.float32)

    pt, lt, lmax = pl.pallas_call(
        _ptable_body,
        in_specs=[pl.BlockSpec(memory_space=pltpu.VMEM)],
        out_specs=[pl.BlockSpec(memory_space=pltpu.VMEM)] * 3,
        out_shape=[
            jax.ShapeDtypeStruct((_T, _C), jnp.int32),
            jax.ShapeDtypeStruct((1, _C), jnp.float32),
            jax.ShapeDtypeStruct((1, 1), jnp.int32),
        ],
    )(spk)

    wqp = jnp.concatenate([W_sq, W_cq, W_sp, W_cp], axis=0)
    bqp = jnp.concatenate([b_sq, b_cq, b_sp, b_cp]).reshape(1, -1)

    mem = pl.pallas_call(
        _fused_body,
        in_specs=[
            pl.BlockSpec(memory_space=pltpu.VMEM),
            pl.BlockSpec(memory_space=pltpu.VMEM),
            pl.BlockSpec(memory_space=pltpu.VMEM),
            pl.BlockSpec(memory_space=pltpu.VMEM),
            pl.BlockSpec(memory_space=pltpu.VMEM),
            pl.BlockSpec(memory_space=pltpu.VMEM),
            pl.BlockSpec(memory_space=pltpu.VMEM),
            pl.BlockSpec(memory_space=pltpu.VMEM),
            pl.BlockSpec(memory_space=pltpu.SMEM),
            pl.BlockSpec(memory_space=pltpu.SMEM),
        ],
        out_specs=pl.BlockSpec(memory_space=pltpu.VMEM),
        out_shape=jax.ShapeDtypeStruct((_C, _D), jnp.float32),
        scratch_shapes=[
            pltpu.VMEM((_B * _T, 3 * _D), jnp.float32),
            pltpu.VMEM((_C, 3 * _D), jnp.float32),
        ],
    )(x, wqp, bqp, W_ih, row2(b_ih), W_hh, row2(b_hh),
      lt.reshape(_C, 1), pt, lmax)

    return mem.reshape(_B, _S, _D)


# speedup vs baseline: 173.4251x; 1.0385x over previous
import jax
import jax.numpy as jnp
from jax import lax
from jax.experimental import pallas as pl
from jax.experimental.pallas import tpu as pltpu

_B, _T, _D, _S, _K = 4, 256, 768, 8, 5
_C = _B * _S
_NEG = -3e38


def _dot_t(x, w):
    return lax.dot_general(x, w, (((1,), (1,)), ((), ())),
                           preferred_element_type=jnp.float32)


def _ptable_body(spk_ref, pt_ref, l_ref, lmax_ref):
    t_row = lax.broadcasted_iota(jnp.int32, (1, _T), 1)
    upper = jnp.where(
        lax.broadcasted_iota(jnp.int32, (_T, _T), 0) <
        lax.broadcasted_iota(jnp.int32, (_T, _T), 1), 1.0, 0.0)
    r_iota = lax.broadcasted_iota(jnp.int32, (_T, 1), 0)
    s_iota = lax.broadcasted_iota(jnp.int32, (_S, 1), 0)

    pts, lts = [], []
    for b in range(_B):
        spk_row = spk_ref[b:b + 1, :]
        o_b = jnp.where((s_iota == spk_row) & (t_row > 0), 1.0, 0.0)
        cnt = jax.lax.dot(o_b, upper,
                          preferred_element_type=jnp.float32)
        rank = jnp.sum(o_b * cnt, axis=0, keepdims=True)
        r_oh = jnp.where(r_iota.astype(jnp.float32) == rank, 1.0, 0.0)
        a_b = o_b * t_row.astype(jnp.float32)
        pts.append(_dot_t(r_oh, a_b))
        lts.append(_dot_t(jnp.ones((1, _T), jnp.float32), o_b))
    pt_ref[...] = jnp.concatenate(pts, axis=1).astype(jnp.int32)
    lt = jnp.concatenate(lts, axis=1)
    l_ref[...] = lt
    lmax_ref[...] = jnp.max(lt, axis=1, keepdims=True).astype(jnp.int32)


def _fused_body(x_ref, wqp_ref, bqp_ref, wih_ref, bih_ref, whh_ref, bhh_ref,
                l_ref, pt_ref, lmax_ref, out_ref, gi_ref, xg_ref):
    row = lax.broadcasted_iota(jnp.int32, (_T, _T), 0)
    col = lax.broadcasted_iota(jnp.int32, (_T, _T), 1)
    causal_bias = jnp.where(col < row, 0.0, _NEG)
    t_col = row[:, :1].astype(jnp.float32)
    cnt = jnp.clip(t_col, 1.0, float(_K))

    for b in range(_B):
        x = x_ref[b]
        qp = _dot_t(x, wqp_ref[...]) + bqp_ref[...]
        scores = _dot_t(qp[:, :2 * _D], qp[:, 2 * _D:])
        masked = scores + causal_bias

        wsel = jnp.zeros((_T, _T), jnp.float32)
        for r in range(_K):
            selj = jnp.argmax(masked, axis=1).reshape(_T, 1)
            pick = col == selj
            wsel = wsel + jnp.where(
                pick & (t_col > float(r)), 1.0, 0.0)
            masked = jnp.where(pick, _NEG, masked)

        summary = jax.lax.dot(wsel, x,
                              preferred_element_type=jnp.float32) / cnt
        gi_ref[_T * b:_T * (b + 1), :] = (
            _dot_t(summary, wih_ref[...]) + bih_ref[...])

    whh = whh_ref[...]
    bhh = bhh_ref[...]
    lcol = l_ref[...]
    lmax = lmax_ref[0, 0]

    for c in range(_C):
        xg_ref[pl.ds(c, 1), :] = gi_ref[
            pl.ds((c // _S) * _T + pt_ref[0, c], 1), :]

    def step(i, h):
        nxt = ((i + 1) % 2) * _C
        for c in range(_C):
            xg_ref[pl.ds(nxt + c, 1), :] = gi_ref[
                pl.ds((c // _S) * _T + pt_ref[i + 1, c], 1), :]
        git = xg_ref[pl.ds((i % 2) * _C, _C), :]
        gh = _dot_t(h, whh) + bhh
        r = jax.nn.sigmoid(git[:, :_D] + gh[:, :_D])
        z = jax.nn.sigmoid(git[:, _D:2 * _D] + gh[:, _D:2 * _D])
        n = jnp.tanh(git[:, 2 * _D:] + r * gh[:, 2 * _D:])
        hn = (1.0 - z) * n + z * h
        m = jnp.where(i.astype(jnp.float32) < lcol, 1.0, 0.0)
        return m * hn + (1.0 - m) * h

    h = lax.fori_loop(0, lmax, step, jnp.zeros((_C, _D), jnp.float32))
    out_ref[...] = h


def kernel(edu_reps, speaker_ids, batch_speaker_maps, W_sq, b_sq, W_cq, b_cq,
           W_sp, b_sp, W_cp, b_cp, W_ih, W_hh, b_ih, b_hh):
    del batch_speaker_maps
    x = edu_reps.astype(jnp.float32)
    spk = speaker_ids.astype(jnp.int32)
    row2 = lambda v: v.reshape(1, -1).astype(jnp.float32)

    pt, lt, lmax = pl.pallas_call(
        _ptable_body,
        in_specs=[pl.BlockSpec(memory_space=pltpu.VMEM)],
        out_specs=[pl.BlockSpec(memory_space=pltpu.VMEM)] * 3,
        out_shape=[
            jax.ShapeDtypeStruct((_T, _C), jnp.int32),
            jax.ShapeDtypeStruct((1, _C), jnp.float32),
            jax.ShapeDtypeStruct((1, 1), jnp.int32),
        ],
    )(spk)

    wqp = jnp.concatenate([W_sq, W_cq, W_sp, W_cp], axis=0)
    bqp = jnp.concatenate([b_sq, b_cq, b_sp, b_cp]).reshape(1, -1)

    mem = pl.pallas_call(
        _fused_body,
        in_specs=[
            pl.BlockSpec(memory_space=pltpu.VMEM),
            pl.BlockSpec(memory_space=pltpu.VMEM),
            pl.BlockSpec(memory_space=pltpu.VMEM),
            pl.BlockSpec(memory_space=pltpu.VMEM),
            pl.BlockSpec(memory_space=pltpu.VMEM),
            pl.BlockSpec(memory_space=pltpu.VMEM),
            pl.BlockSpec(memory_space=pltpu.VMEM),
            pl.BlockSpec(memory_space=pltpu.VMEM),
            pl.BlockSpec(memory_space=pltpu.SMEM),
            pl.BlockSpec(memory_space=pltpu.SMEM),
        ],
        out_specs=pl.BlockSpec(memory_space=pltpu.VMEM),
        out_shape=jax.ShapeDtypeStruct((_C, _D), jnp.float32),
        scratch_shapes=[
            pltpu.VMEM((_B * _T, 3 * _D), jnp.float32),
            pltpu.VMEM((2 * _C, 3 * _D), jnp.float32),
        ],
    )(x, wqp, bqp, W_ih, row2(b_ih), W_hh, row2(b_hh),
      lt.reshape(_C, 1), pt, lmax)

    return mem.reshape(_B, _S, _D)
